# Initial kernel scaffold; baseline (speedup 1.0000x reference)
import jax
import jax.numpy as jnp
from jax.experimental import pallas as pl


def kernel(x, W):
    raise NotImplementedError("write your pallas kernel here")


# speedup vs baseline: 1.1539x; 1.1539x over previous
import jax
import jax.numpy as jnp
from jax import lax
from jax.experimental import pallas as pl
from jax.experimental.pallas import tpu as pltpu
from jax.experimental.pallas import tpu_sc as plsc

_NUM_FIELDS = 26
_VOCAB = 100000
_EMBED = 32
_BATCH = 16384

_NC, _NS, _LANES = 2, 16, 16
_NW = _NC * _NS
_BPW = _BATCH // _NW
_CHUNK = 128
_NCHUNK = _BPW // _CHUNK


def _sc_body(gx_hbm, w_hbm, out_hbm, idx_v, rows_v, acc_v, sem):
    wid = lax.axis_index("s") * _NC + lax.axis_index("c")
    base = wid * _BPW

    zeros = jnp.zeros((_LANES,), jnp.float32)

    def zbody(j, carry):
        acc_v[j, pl.ds(0, _LANES)] = zeros
        acc_v[j, pl.ds(_LANES, _LANES)] = zeros
        return carry

    lax.fori_loop(0, _BPW, zbody, None, unroll=8)

    def fbody(i, carry):
        pltpu.sync_copy(gx_hbm.at[i, pl.ds(wid * _NCHUNK, _NCHUNK)], idx_v)
        descs = [
            pltpu.async_copy(
                w_hbm.at[idx_v.at[c]],
                rows_v.at[pl.ds(c * _CHUNK, _CHUNK)],
                sem,
            )
            for c in range(_NCHUNK)
        ]
        for d in descs:
            d.wait()

        def abody(j, c2):
            plsc.addupdate(acc_v.at[j, pl.ds(0, _LANES)],
                           rows_v[j, pl.ds(0, _LANES)])
            plsc.addupdate(acc_v.at[j, pl.ds(_LANES, _LANES)],
                           rows_v[j, pl.ds(_LANES, _LANES)])
            return c2

        lax.fori_loop(0, _BPW, abody, None, unroll=8)
        return carry

    lax.fori_loop(0, _NUM_FIELDS, fbody, None)

    pltpu.sync_copy(acc_v, out_hbm.at[pl.ds(base, _BPW)])


def kernel(x, W):
    offs = jnp.arange(_NUM_FIELDS, dtype=jnp.int32) * _VOCAB
    gx = (x.T + offs[:, None]).reshape(_NUM_FIELDS, _BATCH // _CHUNK, _CHUNK)
    w_flat = W.reshape(_NUM_FIELDS * _VOCAB, _EMBED)
    mesh = plsc.VectorSubcoreMesh(
        core_axis_name="c", subcore_axis_name="s",
        num_cores=_NC, num_subcores=_NS,
    )
    f = pl.kernel(
        _sc_body,
        out_type=jax.ShapeDtypeStruct((_BATCH, _EMBED), jnp.float32),
        mesh=mesh,
        scratch_types=[
            pltpu.VMEM((_NCHUNK, _CHUNK), jnp.int32),
            pltpu.VMEM((_BPW, _EMBED), jnp.float32),
            pltpu.VMEM((_BPW, _EMBED), jnp.float32),
            pltpu.SemaphoreType.DMA,
        ],
        compiler_params=pltpu.CompilerParams(use_tc_tiling_on_sc=False),
    )
    return f(gx, w_flat)
